# baseline (speedup 1.0000x reference)
import functools

import jax
import jax.numpy as jnp
from jax import lax
from jax.experimental import pallas as pl
from jax.experimental.pallas import tpu as pltpu
from jax.experimental.pallas import tpu_sc as plsc

B = 16
MAX_T = 4096
S_DIM = 128
NA = 4
R = 256
MAXC = B * (MAX_T // R)
NW = 32
KMAX = 8
LN2 = 0.6931471805599453


def _vlog(x):
    bits = lax.bitcast_convert_type(x, jnp.int32)
    ex = lax.shift_right_logical(bits, 23) - 127
    mbits = (bits & 0x007FFFFF) | 0x3F800000
    mm = lax.bitcast_convert_type(mbits, jnp.float32)
    r = (mm - 1.0) / (mm + 1.0)
    r2 = r * r
    p = 1.0 + r2 * (1.0 / 3.0 + r2 * (1.0 / 5.0 + r2 * (1.0 / 7.0
                                                        + r2 / 9.0)))
    return ex.astype(jnp.float32) * LN2 + 2.0 * r * p


def _sc_body(s3, a2, bw, tw, nlw, wq, biasv, out, sbuf, abuf, wv, bv,
             srv, arv, nlv, outv, accv):
    wid = lax.axis_index("s") * 2 + lax.axis_index("c")
    pltpu.sync_copy(bw.at[wid], srv)
    pltpu.sync_copy(tw.at[wid], arv)
    pltpu.sync_copy(nlw.at[wid], nlv)
    pltpu.sync_copy(wq, wv)
    pltpu.sync_copy(biasv, bv)

    lanes = lax.iota(jnp.int32, 16)
    bvec = srv[...]
    tvec = arv[...]
    nvec = nlv[...]
    accv[...] = jnp.zeros((16,), jnp.float32)

    for k in range(KMAX):
        bb = bvec[k]
        t0 = pl.multiple_of(tvec[k], R)
        nl = nvec[k]

        @pl.when(nl > 0)
        def _():
            pltpu.sync_copy(s3.at[bb, pl.ds(t0, R)], sbuf)
            pltpu.sync_copy(a2.at[bb, pl.ds(t0, R)], abuf)
            ngrp = (nl + 15) // 16

            def group(g, acc):
                base = g * 16
                zs = [jnp.zeros((16,), jnp.float32) for _ in range(NA)]
                for r16 in range(16):
                    row = base + r16
                    xq = [sbuf[row, pl.ds(16 * q, 16)] for q in range(8)]
                    for j in range(NA):
                        dotv = xq[0] * wv[j, pl.ds(0, 16)]
                        for q in range(1, 8):
                            dotv = dotv + xq[q] * wv[j, pl.ds(16 * q, 16)]
                        for sh in (1, 2, 4, 8):
                            dotv = dotv + dotv[lanes ^ sh]
                        zs[j] = jnp.where(lanes == r16, dotv, zs[j])
                z = [zs[j] + bv[j] for j in range(NA)]
                m = jnp.maximum(jnp.maximum(z[0], z[1]),
                                jnp.maximum(z[2], z[3]))
                e = (jnp.exp(z[0] - m) + jnp.exp(z[1] - m)
                     + jnp.exp(z[2] - m) + jnp.exp(z[3] - m))
                lse = m + _vlog(e)
                a = abuf[pl.ds(base, 16)]
                taken = jnp.where(a == 0, z[0], z[3])
                taken = jnp.where(a == 1, z[1], taken)
                taken = jnp.where(a == 2, z[2], taken)
                live = (base + lanes) < nl
                return acc + jnp.where(live, lse - taken, 0.0)

            accv[...] = accv[...] + lax.fori_loop(
                0, ngrp, group, jnp.zeros((16,), jnp.float32))

    outv[...] = accv[...]
    pltpu.sync_copy(outv, out.at[wid])


@jax.jit
def _sc_loss(s3, a2, lengths, wq, bias4):
    lengths = lengths.astype(jnp.int32)
    nblk = (lengths + R - 1) // R
    g = jnp.sum(nblk)
    cum = jnp.cumsum(nblk)
    flat = jnp.arange(MAXC, dtype=jnp.int32)
    bid = jnp.searchsorted(cum, flat, side="right").astype(jnp.int32)
    bidc = jnp.minimum(bid, B - 1)
    tid = flat - jnp.where(bid > 0, cum[jnp.maximum(bid - 1, 0)], 0)
    nlive = jnp.where(flat < g, jnp.clip(lengths[bidc] - tid * R, 0, R), 0)
    tid = jnp.clip(tid, 0, MAX_T // R - 1)

    wslot = (flat % NW) * 16 + flat // NW
    bwf = jnp.zeros((NW * 16,), jnp.int32).at[wslot].set(bidc)
    twf = jnp.zeros((NW * 16,), jnp.int32).at[wslot].set(tid * R)
    nlf = jnp.zeros((NW * 16,), jnp.int32).at[wslot].set(nlive)
    biasv = jnp.broadcast_to(bias4[:, None], (NA, 16)).astype(jnp.float32)

    mesh = plsc.VectorSubcoreMesh(core_axis_name="c", subcore_axis_name="s")
    out = functools.partial(
        pl.kernel,
        mesh=mesh,
        out_type=jax.ShapeDtypeStruct((NW, 16), jnp.float32),
        scratch_types=[
            pltpu.VMEM((R, S_DIM), jnp.float32),
            pltpu.VMEM((R,), jnp.int32),
            pltpu.VMEM((NA, S_DIM), jnp.float32),
            pltpu.VMEM((NA, 16), jnp.float32),
            pltpu.VMEM((16,), jnp.int32),
            pltpu.VMEM((16,), jnp.int32),
            pltpu.VMEM((16,), jnp.int32),
            pltpu.VMEM((16,), jnp.float32),
            pltpu.VMEM((16,), jnp.float32),
        ],
    )(_sc_body)(s3, a2, bwf.reshape(NW, 16), twf.reshape(NW, 16),
                nlf.reshape(NW, 16), wq, biasv)
    return jnp.sum(out)


def kernel(s_i_batch, actions_batch, lengths, W, bias, W_stop, W_start):
    del W_stop, W_start
    a2 = actions_batch.astype(jnp.int32)
    wq = W[:, :NA].T.astype(jnp.float32)
    bias4 = bias[:NA].astype(jnp.float32)
    return _sc_loss(s_i_batch, a2, lengths, wq, bias4)


# speedup vs baseline: 1.1352x; 1.1352x over previous
import functools

import jax
import jax.numpy as jnp
from jax import lax
from jax.experimental import pallas as pl
from jax.experimental.pallas import tpu as pltpu
from jax.experimental.pallas import tpu_sc as plsc

TC_SPLIT = 12
B = 16
MAX_T = 4096
S_DIM = 128
NA = 4


TCB = 16


BT = 2048
TNT = MAX_T // BT
MAXG = B * TNT
NBUF = 4


def _body(g2_ref, nlive_ref, bid_ref, tid_ref, s_ref, a_ref, wp_ref, bias_ref,
          out_ref, sbuf, abuf, sem_s, sem_a):
    g2 = g2_ref[0]

    def copies(i, slot):
        b = bid_ref[i]
        t = tid_ref[i]
        c1 = pltpu.make_async_copy(
            s_ref.at[b, pl.ds(t * BT, BT), :], sbuf.at[slot], sem_s.at[slot])
        c2 = pltpu.make_async_copy(
            a_ref.at[b, t], abuf.at[slot], sem_a.at[slot])
        return c1, c2

    def start(i, pr):
        @pl.when(i < g2)
        def _():
            b = bid_ref[i]
            t = tid_ref[i]
            slot = jax.lax.rem(i, NBUF)
            pltpu.async_copy(s_ref.at[b, pl.ds(t * BT, BT), :], sbuf.at[slot],
                             sem_s.at[slot], priority=pr)
            pltpu.async_copy(a_ref.at[b, t], abuf.at[slot], sem_a.at[slot],
                             priority=pr)

    def chunk_contrib(i, slot):
        c1, c2 = copies(i, slot)
        c1.wait()
        c2.wait()
        x = sbuf[slot].astype(jnp.bfloat16)
        z = jnp.dot(x, wp_ref[...], preferred_element_type=jnp.float32)
        zt = z.T[:NA] + bias_ref[...]
        m = jnp.max(zt, axis=0, keepdims=True)
        e = jnp.sum(jnp.exp(zt - m), axis=0, keepdims=True)
        lse = m + jnp.log(e)
        a = abuf[slot]
        taken = jnp.where(a == 0, zt[0:1], 0.0)
        for j in range(1, NA):
            taken += jnp.where(a == j, zt[j:j + 1], 0.0)
        lane = jax.lax.broadcasted_iota(jnp.int32, (1, BT), 1)
        live = lane < nlive_ref[i]
        return jnp.where(live, lse - taken, 0.0)

    for k in range(2):
        start(k, k % 2)

    def step(p, acc):
        i = 2 * p
        start(i + 2, 0)
        start(i + 3, 1)
        slot = jax.lax.rem(i, NBUF)
        acc = acc + chunk_contrib(i, slot)
        acc = acc + chunk_contrib(i + 1, slot + 1)
        return acc

    out_ref[...] = jax.lax.fori_loop(
        0, g2 // 2, step, jnp.zeros((1, BT), jnp.float32))


@jax.jit
def _tc_loss(s, actions4, lengths, wp, bias_col):
    lengths = lengths.astype(jnp.int32)
    nblk = (lengths + BT - 1) // BT
    g = jnp.sum(nblk)
    g2 = g + (g & 1)
    cum = jnp.cumsum(nblk)
    flat = jnp.arange(MAXG, dtype=jnp.int32)
    bid = jnp.searchsorted(cum, flat, side="right").astype(jnp.int32)
    bidc = jnp.minimum(bid, TCB - 1)
    tid = flat - jnp.where(bid > 0, cum[jnp.maximum(bid - 1, 0)], 0)
    nlive = jnp.where(flat < g, jnp.clip(lengths[bidc] - tid * BT, 0, BT), 0)
    tid = jnp.clip(tid, 0, TNT - 1)

    grid_spec = pltpu.PrefetchScalarGridSpec(
        num_scalar_prefetch=4,
        grid=(1,),
        in_specs=[
            pl.BlockSpec(memory_space=pltpu.MemorySpace.HBM),
            pl.BlockSpec(memory_space=pltpu.MemorySpace.HBM),
            pl.BlockSpec((S_DIM, 8), lambda i, *_: (0, 0)),
            pl.BlockSpec((NA, 1), lambda i, *_: (0, 0)),
        ],
        out_specs=pl.BlockSpec((1, BT), lambda i, *_: (0, 0)),
        scratch_shapes=[
            pltpu.VMEM((NBUF, BT, S_DIM), jnp.float32),
            pltpu.VMEM((NBUF, 1, BT), jnp.int32),
            pltpu.SemaphoreType.DMA((NBUF,)),
            pltpu.SemaphoreType.DMA((NBUF,)),
        ],
    )
    out = pl.pallas_call(
        _body,
        grid_spec=grid_spec,
        out_shape=jax.ShapeDtypeStruct((1, BT), jnp.float32),
    )(g2.reshape(1), nlive, bidc, tid, s, actions4, wp, bias_col)
    return jnp.sum(out)


import functools

import jax
import jax.numpy as jnp
from jax import lax
from jax.experimental import pallas as pl
from jax.experimental.pallas import tpu as pltpu
from jax.experimental.pallas import tpu_sc as plsc

B = 16
MAX_T = 4096
S_DIM = 128
NA = 4
R = 256
MAXC = B * (MAX_T // R)
NW = 32
KMAX = 8
LN2 = 0.6931471805599453


def _vlog(x):
    bits = lax.bitcast_convert_type(x, jnp.int32)
    ex = lax.shift_right_logical(bits, 23) - 127
    mbits = (bits & 0x007FFFFF) | 0x3F800000
    mm = lax.bitcast_convert_type(mbits, jnp.float32)
    r = (mm - 1.0) / (mm + 1.0)
    r2 = r * r
    p = 1.0 + r2 * (1.0 / 3.0 + r2 * (1.0 / 5.0 + r2 * (1.0 / 7.0
                                                        + r2 / 9.0)))
    return ex.astype(jnp.float32) * LN2 + 2.0 * r * p


def _sc_body(s3, a2, bw, tw, nlw, wq, biasv, out, sbuf, abuf, wv, bv,
             srv, arv, nlv, outv, accv):
    wid = lax.axis_index("s") * 2 + lax.axis_index("c")
    pltpu.sync_copy(bw.at[wid], srv)
    pltpu.sync_copy(tw.at[wid], arv)
    pltpu.sync_copy(nlw.at[wid], nlv)
    pltpu.sync_copy(wq, wv)
    pltpu.sync_copy(biasv, bv)

    lanes = lax.iota(jnp.int32, 16)
    bvec = srv[...]
    tvec = arv[...]
    nvec = nlv[...]
    accv[...] = jnp.zeros((16,), jnp.float32)

    for k in range(KMAX):
        bb = bvec[k]
        t0 = pl.multiple_of(tvec[k], R)
        nl = nvec[k]

        @pl.when(nl > 0)
        def _():
            pltpu.sync_copy(s3.at[bb, pl.ds(t0, R)], sbuf)
            pltpu.sync_copy(a2.at[bb, pl.ds(t0, R)], abuf)
            ngrp = (nl + 15) // 16

            def group(g, acc):
                base = g * 16
                zs = [jnp.zeros((16,), jnp.float32) for _ in range(NA)]
                for r16 in range(16):
                    row = base + r16
                    xq = [sbuf[row, pl.ds(16 * q, 16)] for q in range(8)]
                    for j in range(NA):
                        dotv = xq[0] * wv[j, pl.ds(0, 16)]
                        for q in range(1, 8):
                            dotv = dotv + xq[q] * wv[j, pl.ds(16 * q, 16)]
                        for sh in (1, 2, 4, 8):
                            dotv = dotv + dotv[lanes ^ sh]
                        zs[j] = jnp.where(lanes == r16, dotv, zs[j])
                z = [zs[j] + bv[j] for j in range(NA)]
                m = jnp.maximum(jnp.maximum(z[0], z[1]),
                                jnp.maximum(z[2], z[3]))
                e = (jnp.exp(z[0] - m) + jnp.exp(z[1] - m)
                     + jnp.exp(z[2] - m) + jnp.exp(z[3] - m))
                lse = m + _vlog(e)
                a = abuf[pl.ds(base, 16)]
                taken = jnp.where(a == 0, z[0], z[3])
                taken = jnp.where(a == 1, z[1], taken)
                taken = jnp.where(a == 2, z[2], taken)
                live = (base + lanes) < nl
                return acc + jnp.where(live, lse - taken, 0.0)

            accv[...] = accv[...] + lax.fori_loop(
                0, ngrp, group, jnp.zeros((16,), jnp.float32))

    outv[...] = accv[...]
    pltpu.sync_copy(outv, out.at[wid])


@jax.jit
def _sc_loss(s3, a2, lengths, wq, bias4):
    lengths = lengths.astype(jnp.int32)
    nblk = (lengths + R - 1) // R
    g = jnp.sum(nblk)
    cum = jnp.cumsum(nblk)
    flat = jnp.arange(MAXC, dtype=jnp.int32)
    bid = jnp.searchsorted(cum, flat, side="right").astype(jnp.int32)
    bidc = jnp.minimum(bid, B - 1)
    tid = flat - jnp.where(bid > 0, cum[jnp.maximum(bid - 1, 0)], 0)
    nlive = jnp.where(flat < g, jnp.clip(lengths[bidc] - tid * R, 0, R), 0)
    tid = jnp.clip(tid, 0, MAX_T // R - 1)

    wslot = (flat % NW) * 16 + flat // NW
    bwf = jnp.zeros((NW * 16,), jnp.int32).at[wslot].set(bidc)
    twf = jnp.zeros((NW * 16,), jnp.int32).at[wslot].set(tid * R)
    nlf = jnp.zeros((NW * 16,), jnp.int32).at[wslot].set(nlive)
    biasv = jnp.broadcast_to(bias4[:, None], (NA, 16)).astype(jnp.float32)

    mesh = plsc.VectorSubcoreMesh(core_axis_name="c", subcore_axis_name="s")
    out = functools.partial(
        pl.kernel,
        mesh=mesh,
        out_type=jax.ShapeDtypeStruct((NW, 16), jnp.float32),
        scratch_types=[
            pltpu.VMEM((R, S_DIM), jnp.float32),
            pltpu.VMEM((R,), jnp.int32),
            pltpu.VMEM((NA, S_DIM), jnp.float32),
            pltpu.VMEM((NA, 16), jnp.float32),
            pltpu.VMEM((16,), jnp.int32),
            pltpu.VMEM((16,), jnp.int32),
            pltpu.VMEM((16,), jnp.int32),
            pltpu.VMEM((16,), jnp.float32),
            pltpu.VMEM((16,), jnp.float32),
        ],
    )(_sc_body)(s3, a2, bwf.reshape(NW, 16), twf.reshape(NW, 16),
                nlf.reshape(NW, 16), wq, biasv)
    return jnp.sum(out)


def kernel(s_i_batch, actions_batch, lengths, W, bias, W_stop, W_start):
    del W_stop, W_start
    lengths = lengths.astype(jnp.int32)
    bidx = jnp.arange(B, dtype=jnp.int32)
    len_tc = jnp.where(bidx < TC_SPLIT, lengths, 0)
    len_sc = jnp.where(bidx >= TC_SPLIT, lengths, 0)

    wp = jnp.zeros((S_DIM, 8), jnp.bfloat16).at[:, :NA].set(
        W[:, :NA].astype(jnp.bfloat16))
    bias_col = bias[:NA].reshape(NA, 1)
    actions4 = actions_batch.astype(jnp.int32).reshape(B, TNT, 1, BT)
    tc = _tc_loss(s_i_batch, actions4, len_tc, wp, bias_col)

    a2 = actions_batch.astype(jnp.int32)
    wq = W[:, :NA].T.astype(jnp.float32)
    bias4 = bias[:NA].astype(jnp.float32)
    sc = _sc_loss(s_i_batch, a2, len_sc, wq, bias4)
    return tc + sc


# speedup vs baseline: 3.0253x; 2.6649x over previous
import jax
import jax.numpy as jnp
from jax.experimental import pallas as pl
from jax.experimental.pallas import tpu as pltpu

B = 16
MAX_T = 4096
S_DIM = 128
NA = 4
BT = 2048
NT = MAX_T // BT
MAXG = B * NT
NBUF = 4


def _body(g2_ref, nlive_ref, bid_ref, tid_ref, s_ref, a_ref, wp_ref, bias_ref,
          out_ref, sbuf, abuf, sem_s, sem_a):
    g2 = g2_ref[0]

    def copies(i, slot):
        b = bid_ref[i]
        t = tid_ref[i]
        c1 = pltpu.make_async_copy(
            s_ref.at[b, pl.ds(t * BT, BT), :], sbuf.at[slot], sem_s.at[slot])
        c2 = pltpu.make_async_copy(
            a_ref.at[b, t], abuf.at[slot], sem_a.at[slot])
        return c1, c2

    def start(i, pr):
        @pl.when(i < g2)
        def _():
            b = bid_ref[i]
            t = tid_ref[i]
            slot = jax.lax.rem(i, NBUF)
            pltpu.async_copy(s_ref.at[b, pl.ds(t * BT, BT), :], sbuf.at[slot],
                             sem_s.at[slot], priority=pr)
            pltpu.async_copy(a_ref.at[b, t], abuf.at[slot], sem_a.at[slot],
                             priority=pr)

    def chunk_contrib(i, slot):
        c1, c2 = copies(i, slot)
        c1.wait()
        c2.wait()
        x = sbuf[slot].astype(jnp.bfloat16)
        z = jnp.dot(x, wp_ref[...], preferred_element_type=jnp.float32)
        zt = z.T[:NA] + bias_ref[...]
        m = jnp.max(zt, axis=0, keepdims=True)
        e = jnp.sum(jnp.exp(zt - m), axis=0, keepdims=True)
        lse = m + jnp.log(e)
        a = abuf[slot]
        taken = jnp.where(a == 0, zt[0:1], 0.0)
        for j in range(1, NA):
            taken += jnp.where(a == j, zt[j:j + 1], 0.0)
        lane = jax.lax.broadcasted_iota(jnp.int32, (1, BT), 1)
        live = lane < nlive_ref[i]
        return jnp.where(live, lse - taken, 0.0)

    for k in range(2):
        start(k, k % 2)

    def step(p, acc):
        i = 2 * p
        start(i + 2, 0)
        start(i + 3, 1)
        slot = jax.lax.rem(i, NBUF)
        acc = acc + chunk_contrib(i, slot)
        acc = acc + chunk_contrib(i + 1, slot + 1)
        return acc

    out_ref[...] = jax.lax.fori_loop(
        0, g2 // 2, step, jnp.zeros((1, BT), jnp.float32))


@jax.jit
def _tc_loss(s, actions4, lengths, wp, bias_col):
    lengths = lengths.astype(jnp.int32)
    nblk = (lengths + BT - 1) // BT
    g = jnp.sum(nblk)
    g2 = g + (g & 1)
    cum = jnp.cumsum(nblk)
    flat = jnp.arange(MAXG, dtype=jnp.int32)
    bid = jnp.searchsorted(cum, flat, side="right").astype(jnp.int32)
    bidc = jnp.minimum(bid, B - 1)
    tid = flat - jnp.where(bid > 0, cum[jnp.maximum(bid - 1, 0)], 0)
    nlive = jnp.where(flat < g, jnp.clip(lengths[bidc] - tid * BT, 0, BT), 0)
    tid = jnp.clip(tid, 0, NT - 1)

    grid_spec = pltpu.PrefetchScalarGridSpec(
        num_scalar_prefetch=4,
        grid=(1,),
        in_specs=[
            pl.BlockSpec(memory_space=pltpu.MemorySpace.HBM),
            pl.BlockSpec(memory_space=pltpu.MemorySpace.HBM),
            pl.BlockSpec((S_DIM, 8), lambda i, *_: (0, 0)),
            pl.BlockSpec((NA, 1), lambda i, *_: (0, 0)),
        ],
        out_specs=pl.BlockSpec((1, BT), lambda i, *_: (0, 0)),
        scratch_shapes=[
            pltpu.VMEM((NBUF, BT, S_DIM), jnp.float32),
            pltpu.VMEM((NBUF, 1, BT), jnp.int32),
            pltpu.SemaphoreType.DMA((NBUF,)),
            pltpu.SemaphoreType.DMA((NBUF,)),
        ],
    )
    out = pl.pallas_call(
        _body,
        grid_spec=grid_spec,
        out_shape=jax.ShapeDtypeStruct((1, BT), jnp.float32),
    )(g2.reshape(1), nlive, bidc, tid, s, actions4, wp, bias_col)
    return jnp.sum(out)


def kernel(s_i_batch, actions_batch, lengths, W, bias, W_stop, W_start):
    del W_stop, W_start
    wp = jnp.zeros((S_DIM, 8), jnp.bfloat16).at[:, :NA].set(
        W[:, :NA].astype(jnp.bfloat16))
    bias_col = bias[:NA].reshape(NA, 1)
    actions4 = actions_batch.astype(jnp.int32).reshape(B, NT, 1, BT)
    return _tc_loss(s_i_batch, actions4, lengths, wp, bias_col)
